# baseline (speedup 1.0000x reference)
import functools

import jax
import jax.numpy as jnp
from jax import lax
from jax.experimental import pallas as pl
from jax.experimental.pallas import tpu as pltpu
from jax.experimental.pallas import tpu_sc as plsc

_D = 64
_S = 200
_B = 4096
_W = 128
_T = 512
_NBT = _T // _W


@functools.lru_cache(maxsize=None)
def _build(V: int):
    info = plsc.get_sparse_core_info()
    nc, ns = info.num_cores, info.num_subcores
    nw = nc * ns
    ntok = _S * _B
    tasks = ntok // _T
    t_per_w = tasks // nw
    tok_per_w = ntok // nw
    asm_words = 8 * _NBT * 8 * _W
    dt_span = _NBT * 8 * _W
    mesh = plsc.VectorSubcoreMesh(core_axis_name="c", subcore_axis_name="s")

    @functools.partial(
        pl.kernel,
        mesh=mesh,
        out_type=jax.ShapeDtypeStruct((ntok * _D,), jnp.float32),
        scratch_types=[
            pltpu.VMEM((tok_per_w,), jnp.int32),
            pltpu.VMEM((_W, _D), jnp.float32),
            pltpu.VMEM((_W, _D), jnp.float32),
            pltpu.VMEM((asm_words,), jnp.float32),
            pltpu.VMEM((asm_words,), jnp.float32),
            pltpu.SemaphoreType.DMA,
            pltpu.SemaphoreType.DMA,
            pltpu.SemaphoreType.DMA,
            pltpu.SemaphoreType.DMA,
        ],
        compiler_params=pltpu.CompilerParams(
            use_tc_tiling_on_sc=False, needs_layout_passes=False
        ),
    )
    def emb_kernel(idx_hbm, table_hbm, out_hbm, idx_v, rb0, rb1, as0, as1,
                   g0, g1, s0, s1):
        rbufs = (rb0, rb1)
        asms = (as0, as1)
        gsems = (g0, g1)
        ssems = (s0, s1)
        wid = lax.axis_index("s") * nc + lax.axis_index("c")
        tbase = wid * t_per_w
        pltpu.sync_copy(idx_hbm.at[pl.ds(wid * tok_per_w, tok_per_w)], idx_v)
        iota16 = lax.iota(jnp.int32, 16)

        def start_gather(j, btl, slot):
            off = j * _T + btl * _W
            pltpu.async_copy(
                table_hbm.at[idx_v.at[pl.ds(off, _W)]],
                rbufs[slot], gsems[slot],
            )

        def drain_gather(slot):
            pltpu.make_async_copy(
                table_hbm.at[pl.ds(0, _W)], rbufs[slot], gsems[slot]
            ).wait()

        def drain_stores(slot):
            pltpu.make_async_copy(
                out_hbm.at[pl.ds(0, asm_words)], asms[slot], ssems[slot]
            ).wait()

        def task(j, slot):
            t = tbase + j
            s = t >> 3
            oct_ = t & 7
            asm = asms[slot]
            for btl in range(_NBT):
                gslot = btl & 1
                drain_gather(gslot)
                rb = rbufs[gslot]

                def dbody(d, carry, rb=rb, btl=btl):
                    obase = ((d >> 3) * dt_span + btl * (8 * _W)
                             + (d & 7) * _W)
                    col16 = jnp.broadcast_to(d, (16,)).astype(jnp.int32)
                    for lg in range(_W // 16):
                        v = plsc.load_gather(rb, [iota16 + lg * 16, col16])
                        asm[pl.ds(obase + lg * 16, 16)] = v
                    return carry

                lax.fori_loop(0, _D, dbody, 0)
                if btl < _NBT - 2:
                    start_gather(j, btl + 2, gslot)
                else:

                    @pl.when(j + 1 < t_per_w)
                    def _(j=j, btl=btl, gslot=gslot):
                        start_gather(j + 1, btl - (_NBT - 2), gslot)
            obase_o = s * (8 * 32 * 8 * _W) + oct_ * dt_span
            for dt in range(8):
                pltpu.async_copy(
                    asm.at[pl.ds(dt * dt_span, dt_span)],
                    out_hbm.at[pl.ds(obase_o + dt * (32 * 8 * _W), dt_span)],
                    ssems[slot],
                )

        start_gather(0, 0, 0)
        start_gather(0, 1, 1)

        def body(k2, carry):
            for b2 in range(2):
                j = 2 * k2 + b2

                @pl.when(j >= 2)
                def _(b2=b2):
                    drain_stores(b2)

                task(j, b2)
            return carry

        lax.fori_loop(0, t_per_w // 2, body, 0)
        drain_stores(0)
        drain_stores(1)

    return emb_kernel


def kernel(token_ids, emb):
    bsz, seq = token_ids.shape
    idx = jnp.transpose(token_ids).reshape(-1).astype(jnp.int32)
    o = _build(emb.shape[0])(idx, emb)
    o5 = o.reshape(seq, 8, bsz // 128, 8, 128)
    t = jnp.transpose(o5, (2, 4, 0, 1, 3))
    return t.reshape(bsz, seq, _D)


# speedup vs baseline: 1.3919x; 1.3919x over previous
import functools

import jax
import jax.numpy as jnp
from jax import lax
from jax.experimental import pallas as pl
from jax.experimental.pallas import tpu as pltpu
from jax.experimental.pallas import tpu_sc as plsc

_D = 64
_S = 200
_B = 4096
_W = 128
_T = 512
_NBT = _T // _W


@functools.lru_cache(maxsize=None)
def _build(V: int):
    info = plsc.get_sparse_core_info()
    nc, ns = info.num_cores, info.num_subcores
    nw = nc * ns
    ntok = _S * _B
    tasks = ntok // _T
    t_per_w = tasks // nw
    tok_per_w = ntok // nw
    asm_words = 8 * _NBT * 8 * _W
    dt_span = _NBT * 8 * _W
    mesh = plsc.VectorSubcoreMesh(core_axis_name="c", subcore_axis_name="s")

    @functools.partial(
        pl.kernel,
        mesh=mesh,
        out_type=jax.ShapeDtypeStruct((ntok * _D,), jnp.float32),
        scratch_types=[
            pltpu.VMEM((tok_per_w,), jnp.int32),
            pltpu.VMEM((_W, _D), jnp.float32),
            pltpu.VMEM((_W, _D), jnp.float32),
            pltpu.VMEM((asm_words,), jnp.float32),
            pltpu.VMEM((asm_words,), jnp.float32),
            pltpu.SemaphoreType.DMA,
            pltpu.SemaphoreType.DMA,
            pltpu.SemaphoreType.DMA,
            pltpu.SemaphoreType.DMA,
        ],
        compiler_params=pltpu.CompilerParams(
            use_tc_tiling_on_sc=False, needs_layout_passes=False
        ),
    )
    def emb_kernel(idx_hbm, table_hbm, out_hbm, idx_v, rb0, rb1, as0, as1,
                   g0, g1, s0, s1):
        rbufs = (rb0, rb1)
        asms = (as0, as1)
        gsems = (g0, g1)
        ssems = (s0, s1)
        wid = lax.axis_index("s") * nc + lax.axis_index("c")
        tbase = wid * t_per_w
        pltpu.sync_copy(idx_hbm.at[pl.ds(wid * tok_per_w, tok_per_w)], idx_v)
        iota16 = lax.iota(jnp.int32, 16)

        def start_gather(j, btl, slot):
            off = j * _T + btl * _W
            pltpu.async_copy(
                table_hbm.at[idx_v.at[pl.ds(off, _W)]],
                rbufs[slot], gsems[slot],
            )

        def drain_gather(slot):
            pltpu.make_async_copy(
                table_hbm.at[pl.ds(0, _W)], rbufs[slot], gsems[slot]
            ).wait()

        def drain_stores(slot):
            pltpu.make_async_copy(
                out_hbm.at[pl.ds(0, asm_words)], asms[slot], ssems[slot]
            ).wait()

        def task(j, slot):
            t = tbase + j
            s = t >> 3
            oct_ = t & 7
            asm = asms[slot]
            for btl in range(_NBT):
                gslot = btl & 1
                drain_gather(gslot)
                rb = rbufs[gslot]
                bvecs = []
                for lg in range(_D // 16):
                    dvec = iota16 + (lg * 16)
                    bvecs.append(
                        (dvec >> 3) * dt_span + (dvec & 7) * _W
                        + btl * (8 * _W)
                    )

                @plsc.parallel_loop(0, _W, unroll=4)
                def _(b, rb=rb, bvecs=bvecs):
                    for lg in range(_D // 16):
                        v = rb[b, pl.ds(lg * 16, 16)]
                        plsc.store_scatter(asm, [bvecs[lg] + b], v)
                if btl < _NBT - 2:
                    start_gather(j, btl + 2, gslot)
                else:

                    @pl.when(j + 1 < t_per_w)
                    def _(j=j, btl=btl, gslot=gslot):
                        start_gather(j + 1, btl - (_NBT - 2), gslot)
            obase_o = s * (8 * 32 * 8 * _W) + oct_ * dt_span
            for dt in range(8):
                pltpu.async_copy(
                    asm.at[pl.ds(dt * dt_span, dt_span)],
                    out_hbm.at[pl.ds(obase_o + dt * (32 * 8 * _W), dt_span)],
                    ssems[slot],
                )

        start_gather(0, 0, 0)
        start_gather(0, 1, 1)

        def body(k2, carry):
            for b2 in range(2):
                j = 2 * k2 + b2

                @pl.when(j >= 2)
                def _(b2=b2):
                    drain_stores(b2)

                task(j, b2)
            return carry

        lax.fori_loop(0, t_per_w // 2, body, 0)
        drain_stores(0)
        drain_stores(1)

    return emb_kernel


def kernel(token_ids, emb):
    bsz, seq = token_ids.shape
    idx = jnp.transpose(token_ids).reshape(-1).astype(jnp.int32)
    o = _build(emb.shape[0])(idx, emb)
    o5 = o.reshape(seq, 8, bsz // 128, 8, 128)
    t = jnp.transpose(o5, (2, 4, 0, 1, 3))
    return t.reshape(bsz, seq, _D)
